# baseline (speedup 1.0000x reference)
import jax
import jax.numpy as jnp
from jax.experimental import pallas as pl
from jax.experimental.pallas import tpu as pltpu

_CACHE_SIZE = 4096.0 + 16.0
_NUM_KV_HEADS = 8
_NUM_BLOCKS = 2048
_BS = 64
_C = 128
_GH = 4
_NH = _NUM_KV_HEADS // _GH
_NL = _NUM_BLOCKS // _C
_NCH = _NH * _NL
_NBUF = 3


def _src_slice(ref, c):
    hh = jax.lax.rem(c, _NH)
    li = jax.lax.div(c, _NH)
    return ref.at[pl.ds(hh * _GH, _GH), :, :, :, pl.ds(li * _C, _C)]


def _body(sb_ref, pos_ref, x_ref, o_ref, xbuf, obuf, cs_ref, ss_ref, in_sems, out_sems):
    i = pl.program_id(0)

    @pl.when(i == 0)
    def _tables():
        bid = jax.lax.broadcasted_iota(jnp.int32, (_BS, _NUM_BLOCKS), 1)
        barange = jax.lax.broadcasted_iota(jnp.int32, (_BS, _NUM_BLOCKS), 0)
        match = bid == sb_ref[...]
        key = jnp.where(match, barange, -1)
        w = jnp.max(key, axis=0, keepdims=True)
        onehot = jnp.logical_and(barange == w, match)
        posf = pos_ref[...].astype(jnp.float32)
        ev = jnp.maximum(posf - _CACHE_SIZE, 0.0)
        theta = jnp.sum(jnp.where(onehot, ev, 0.0), axis=0, keepdims=True)
        d8i = jax.lax.broadcasted_iota(jnp.int32, (8, 1, 8, _C), 0)
        li = jax.lax.broadcasted_iota(jnp.int32, (8, 1, 8, _C), 2)
        j = (d8i * 8 + li).astype(jnp.float32)
        inv_freq = jnp.exp(j * (-jnp.log(10000.0) / 64.0))
        for li_c in range(_NL):
            th = theta[:, li_c * _C:(li_c + 1) * _C].reshape(1, 1, 1, _C)
            freqs = th * inv_freq
            cs_ref[li_c] = jnp.cos(freqs)
            ss_ref[li_c] = jnp.sin(freqs)

    @pl.when(i < _NCH)
    def _start_in():
        k = jax.lax.rem(i, _NBUF)
        pltpu.make_async_copy(_src_slice(x_ref, i), xbuf.at[k], in_sems.at[k]).start()

    @pl.when(jnp.logical_and(i >= 1, i <= _NCH))
    def _compute():
        c = i - 1
        k = jax.lax.rem(c, _NBUF)
        pltpu.make_async_copy(_src_slice(x_ref, c), xbuf.at[k], in_sems.at[k]).wait()

        @pl.when(c >= _NBUF)
        def _drain():
            pltpu.make_async_copy(obuf.at[k], _src_slice(o_ref, c - _NBUF), out_sems.at[k]).wait()

        li = jax.lax.div(c, _NH)
        cc = cs_ref[li]
        ss = ss_ref[li]
        for h in range(_GH):
            x1 = xbuf[k, h, :8]
            x2 = xbuf[k, h, 8:]
            obuf[k, h, :8] = x1 * cc - x2 * ss
            obuf[k, h, 8:] = x2 * cc + x1 * ss
        pltpu.make_async_copy(obuf.at[k], _src_slice(o_ref, c), out_sems.at[k]).start()

    @pl.when(i == _NCH + 1)
    def _final_drain():
        for c in range(_NCH - _NBUF, _NCH):
            k = c % _NBUF
            pltpu.make_async_copy(obuf.at[k], _src_slice(o_ref, c), out_sems.at[k]).wait()


def kernel(key_cache, block_tables, positions):
    x = jnp.transpose(key_cache, (1, 2, 3, 4, 0))
    sb = block_tables[:, :1]
    pos = positions.reshape(_BS, 1)
    out = pl.pallas_call(
        _body,
        grid=(_NCH + 2,),
        in_specs=[
            pl.BlockSpec((_BS, 1), lambda i: (0, 0)),
            pl.BlockSpec((_BS, 1), lambda i: (0, 0)),
            pl.BlockSpec(memory_space=pl.ANY),
        ],
        out_specs=pl.BlockSpec(memory_space=pl.ANY),
        out_shape=jax.ShapeDtypeStruct((_NUM_KV_HEADS, 16, 16, 8, _NUM_BLOCKS), jnp.float32),
        scratch_shapes=[
            pltpu.VMEM((_NBUF, _GH, 16, 16, 8, _C), jnp.float32),
            pltpu.VMEM((_NBUF, _GH, 16, 16, 8, _C), jnp.float32),
            pltpu.VMEM((_NL, 8, 1, 8, _C), jnp.float32),
            pltpu.VMEM((_NL, 8, 1, 8, _C), jnp.float32),
            pltpu.SemaphoreType.DMA((_NBUF,)),
            pltpu.SemaphoreType.DMA((_NBUF,)),
        ],
        compiler_params=pltpu.CompilerParams(
            dimension_semantics=("arbitrary",),
        ),
    )(sb, pos, x)
    return jnp.transpose(out, (4, 0, 1, 2, 3))


# speedup vs baseline: 1.0200x; 1.0200x over previous
import jax
import jax.numpy as jnp
from jax.experimental import pallas as pl
from jax.experimental.pallas import tpu as pltpu

_CACHE_SIZE = 4096.0 + 16.0
_NUM_KV_HEADS = 8
_NUM_BLOCKS = 2048
_BS = 64
_C = 128
_GH = 8
_NH = _NUM_KV_HEADS // _GH
_NL = _NUM_BLOCKS // _C
_NCH = _NH * _NL
_NBUF = 3


def _src_slice(ref, c):
    hh = jax.lax.rem(c, _NH)
    li = jax.lax.div(c, _NH)
    return ref.at[pl.ds(hh * _GH, _GH), :, :, :, pl.ds(li * _C, _C)]


def _body(sb_ref, pos_ref, x_ref, o_ref, xbuf, obuf, cs_ref, ss_ref, in_sems, out_sems):
    i = pl.program_id(0)

    @pl.when(i == 0)
    def _tables():
        bid = jax.lax.broadcasted_iota(jnp.int32, (_BS, _NUM_BLOCKS), 1)
        barange = jax.lax.broadcasted_iota(jnp.int32, (_BS, _NUM_BLOCKS), 0)
        match = bid == sb_ref[...]
        key = jnp.where(match, barange, -1)
        w = jnp.max(key, axis=0, keepdims=True)
        onehot = jnp.logical_and(barange == w, match)
        posf = pos_ref[...].astype(jnp.float32)
        ev = jnp.maximum(posf - _CACHE_SIZE, 0.0)
        theta = jnp.sum(jnp.where(onehot, ev, 0.0), axis=0, keepdims=True)
        d8i = jax.lax.broadcasted_iota(jnp.int32, (8, 1, 8, _C), 0)
        li = jax.lax.broadcasted_iota(jnp.int32, (8, 1, 8, _C), 2)
        j = (d8i * 8 + li).astype(jnp.float32)
        inv_freq = jnp.exp(j * (-jnp.log(10000.0) / 64.0))
        for li_c in range(_NL):
            th = theta[:, li_c * _C:(li_c + 1) * _C].reshape(1, 1, 1, _C)
            freqs = th * inv_freq
            cs_ref[li_c] = jnp.cos(freqs)
            ss_ref[li_c] = jnp.sin(freqs)

    @pl.when(i < _NCH)
    def _start_in():
        k = jax.lax.rem(i, _NBUF)
        pltpu.make_async_copy(_src_slice(x_ref, i), xbuf.at[k], in_sems.at[k]).start()

    @pl.when(jnp.logical_and(i >= 1, i <= _NCH))
    def _compute():
        c = i - 1
        k = jax.lax.rem(c, _NBUF)
        pltpu.make_async_copy(_src_slice(x_ref, c), xbuf.at[k], in_sems.at[k]).wait()

        @pl.when(c >= _NBUF)
        def _drain():
            pltpu.make_async_copy(obuf.at[k], _src_slice(o_ref, c - _NBUF), out_sems.at[k]).wait()

        li = jax.lax.div(c, _NH)
        cc = cs_ref[li]
        ss = ss_ref[li]
        for h in range(_GH):
            x1 = xbuf[k, h, :8]
            x2 = xbuf[k, h, 8:]
            obuf[k, h, :8] = x1 * cc - x2 * ss
            obuf[k, h, 8:] = x2 * cc + x1 * ss
        pltpu.make_async_copy(obuf.at[k], _src_slice(o_ref, c), out_sems.at[k]).start()

    @pl.when(i == _NCH + 1)
    def _final_drain():
        for c in range(_NCH - _NBUF, _NCH):
            k = c % _NBUF
            pltpu.make_async_copy(obuf.at[k], _src_slice(o_ref, c), out_sems.at[k]).wait()


def kernel(key_cache, block_tables, positions):
    x = jnp.transpose(key_cache, (1, 2, 3, 4, 0))
    sb = block_tables[:, :1]
    pos = positions.reshape(_BS, 1)
    out = pl.pallas_call(
        _body,
        grid=(_NCH + 2,),
        in_specs=[
            pl.BlockSpec((_BS, 1), lambda i: (0, 0)),
            pl.BlockSpec((_BS, 1), lambda i: (0, 0)),
            pl.BlockSpec(memory_space=pl.ANY),
        ],
        out_specs=pl.BlockSpec(memory_space=pl.ANY),
        out_shape=jax.ShapeDtypeStruct((_NUM_KV_HEADS, 16, 16, 8, _NUM_BLOCKS), jnp.float32),
        scratch_shapes=[
            pltpu.VMEM((_NBUF, _GH, 16, 16, 8, _C), jnp.float32),
            pltpu.VMEM((_NBUF, _GH, 16, 16, 8, _C), jnp.float32),
            pltpu.VMEM((_NL, 8, 1, 8, _C), jnp.float32),
            pltpu.VMEM((_NL, 8, 1, 8, _C), jnp.float32),
            pltpu.SemaphoreType.DMA((_NBUF,)),
            pltpu.SemaphoreType.DMA((_NBUF,)),
        ],
        compiler_params=pltpu.CompilerParams(
            dimension_semantics=("arbitrary",),
        ),
    )(sb, pos, x)
    return jnp.transpose(out, (4, 0, 1, 2, 3))


# speedup vs baseline: 1.0204x; 1.0004x over previous
import jax
import jax.numpy as jnp
from jax.experimental import pallas as pl
from jax.experimental.pallas import tpu as pltpu

_CACHE_SIZE = 4096.0 + 16.0
_NUM_KV_HEADS = 8
_NUM_BLOCKS = 2048
_BS = 64
_C = 128
_GH = 8


def _rotate_body(sb_ref, pos_ref, x_ref, o_ref):
    i = pl.program_id(0)
    bid = jax.lax.broadcasted_iota(jnp.int32, (_BS, _C), 1) + i * _C
    barange = jax.lax.broadcasted_iota(jnp.int32, (_BS, _C), 0)
    match = bid == sb_ref[...]
    key = jnp.where(match, barange, -1)
    w = jnp.max(key, axis=0, keepdims=True)
    onehot = jnp.logical_and(barange == w, match)
    posf = pos_ref[...].astype(jnp.float32)
    ev = jnp.maximum(posf - _CACHE_SIZE, 0.0)
    theta = jnp.sum(jnp.where(onehot, ev, 0.0), axis=0, keepdims=True)

    d8i = jax.lax.broadcasted_iota(jnp.int32, (1, 8, 1, 8, _C), 1)
    li = jax.lax.broadcasted_iota(jnp.int32, (1, 8, 1, 8, _C), 3)
    j = (d8i * 8 + li).astype(jnp.float32)
    inv_freq = jnp.exp(j * (-jnp.log(10000.0) / 64.0))
    freqs = theta.reshape(1, 1, 1, 1, _C) * inv_freq
    c = jnp.cos(freqs)
    s = jnp.sin(freqs)

    for h in range(_GH):
        x1 = x_ref[h, :8]
        x2 = x_ref[h, 8:]
        o_ref[h, :8] = x1 * c[0] - x2 * s[0]
        o_ref[h, 8:] = x2 * c[0] + x1 * s[0]


def kernel(key_cache, block_tables, positions):
    x = jnp.transpose(key_cache, (1, 2, 3, 4, 0))
    sb = block_tables[:, :1]
    pos = positions.reshape(_BS, 1)
    out = pl.pallas_call(
        _rotate_body,
        grid=(_NUM_BLOCKS // _C,),
        in_specs=[
            pl.BlockSpec((_BS, 1), lambda i: (0, 0)),
            pl.BlockSpec((_BS, 1), lambda i: (0, 0)),
            pl.BlockSpec((_GH, 16, 16, 8, _C), lambda i: (0, 0, 0, 0, i)),
        ],
        out_specs=pl.BlockSpec((_GH, 16, 16, 8, _C), lambda i: (0, 0, 0, 0, i)),
        out_shape=jax.ShapeDtypeStruct((_NUM_KV_HEADS, 16, 16, 8, _NUM_BLOCKS), jnp.float32),
        compiler_params=pltpu.CompilerParams(
            dimension_semantics=("parallel",),
        ),
    )(sb, pos, x)
    return jnp.transpose(out, (4, 0, 1, 2, 3))
